# baseline (speedup 1.0000x reference)
import functools

import jax
import jax.numpy as jnp
from jax import lax
from jax.experimental import pallas as pl
from jax.experimental.pallas import tpu as pltpu
from jax.experimental.pallas import tpu_sc as plsc


@functools.lru_cache(maxsize=None)
def _make_sc_gather(N, Ec, D, dtype_name):
    dtype = jnp.dtype(dtype_name)
    info = plsc.get_sparse_core_info()
    NC, NS = info.num_cores, info.num_subcores
    NW = NC * NS
    assert Ec % NW == 0
    epw = Ec // NW
    C = next(c for c in range(64, 7, -8) if epw % c == 0)
    n = epw // C
    assert n >= 4
    SROW = next(s for s in (16, 10, 8, 5, 4, 2, 1)
                if N % s == 0 and (N // s) % 8 == 0)
    rpt = N // SROW

    mesh = plsc.VectorSubcoreMesh(core_axis_name="c", subcore_axis_name="s")
    D2 = D // 2
    L = 16

    @functools.partial(
        pl.kernel,
        out_type=(
            jax.ShapeDtypeStruct((Ec, D2), jnp.int32),
            jax.ShapeDtypeStruct((Ec, D2), jnp.int32),
        ),
        mesh=mesh,
        scratch_types=[
            pltpu.VMEM((C,), jnp.int32),
            pltpu.VMEM((C,), jnp.int32),
            pltpu.VMEM((C,), jnp.int32),
            pltpu.VMEM((C,), jnp.int32),
            pltpu.VMEM((C, D), dtype),
            pltpu.VMEM((C, D), dtype),
            pltpu.VMEM((C, D), dtype),
            pltpu.VMEM((C, D), dtype),
            pltpu.VMEM((C, D2), jnp.int32),
            pltpu.VMEM((C, D2), jnp.int32),
            pltpu.VMEM((C, D2), jnp.int32),
            pltpu.VMEM((C, D2), jnp.int32),
            pltpu.VMEM_SHARED((N, D), dtype),
            pltpu.SemaphoreType.DMA,
            pltpu.SemaphoreType.DMA,
            pltpu.SemaphoreType.DMA,
            pltpu.SemaphoreType.DMA,
            pltpu.SemaphoreType.DMA,
            pltpu.SemaphoreType.DMA,
        ],
    )
    def sc_gather(x_hbm, src_hbm, dst_hbm, hs_hbm, hd_hbm,
                  i_s0, i_s1, i_d0, i_d1, r_s0, r_s1, r_d0, r_d1,
                  p_s0, p_s1, p_d0, p_d1, x_sp,
                  sem_i0, sem_i1, sem_g0, sem_g1, sem_v0, sem_v1):
        idx_s = (i_s0, i_s1)
        idx_d = (i_d0, i_d1)
        rows_s = (r_s0, r_s1)
        rows_d = (r_d0, r_d1)
        pk_s = (p_s0, p_s1)
        pk_d = (p_d0, p_d1)
        sem_i = (sem_i0, sem_i1)
        sem_g = (sem_g0, sem_g1)
        sem_v = (sem_v0, sem_v1)
        cid = lax.axis_index("c")
        sid = lax.axis_index("s")
        wid = cid * NS + sid
        base_w = wid * epw

        @pl.when(sid < SROW)
        def _stage():
            pltpu.sync_copy(x_hbm.at[pl.ds(sid * rpt, rpt)],
                            x_sp.at[pl.ds(sid * rpt, rpt)])

        plsc.subcore_barrier()

        def issue_idx(c, p):
            base = base_w + c * C
            pltpu.async_copy(src_hbm.at[pl.ds(base, C)], idx_s[p], sem_i[p])
            pltpu.async_copy(dst_hbm.at[pl.ds(base, C)], idx_d[p], sem_i[p])

        def wait_idx(p):
            pltpu.make_async_copy(
                src_hbm.at[pl.ds(0, C)], idx_s[p], sem_i[p]).wait()
            pltpu.make_async_copy(
                dst_hbm.at[pl.ds(0, C)], idx_d[p], sem_i[p]).wait()

        def issue_gather(p):
            pltpu.async_copy(x_sp.at[idx_s[p]], rows_s[p], sem_g[p])
            pltpu.async_copy(x_sp.at[idx_d[p]], rows_d[p], sem_g[p])

        def wait_gather(p):
            pltpu.make_async_copy(
                x_sp.at[idx_s[p]], rows_s[p], sem_g[p]).wait()
            pltpu.make_async_copy(
                x_sp.at[idx_d[p]], rows_d[p], sem_g[p]).wait()

        def issue_scatter(c, p):
            base = base_w + c * C
            pltpu.async_copy(pk_s[p], hs_hbm.at[pl.ds(base, C)], sem_v[p])
            pltpu.async_copy(pk_d[p], hd_hbm.at[pl.ds(base, C)], sem_v[p])

        def wait_scatter(p):
            pltpu.make_async_copy(
                pk_s[p], hs_hbm.at[pl.ds(0, C)], sem_v[p]).wait()
            pltpu.make_async_copy(
                pk_d[p], hd_hbm.at[pl.ds(0, C)], sem_v[p]).wait()

        def pack_rows(p):
            half = jnp.uint32(0x7FFF)
            hmask = jnp.uint32(0xFFFF0000)
            one = jnp.uint32(1)

            def row(r, carry):
                for g in range(D2 // L):
                    sl = pl.ds(L * g, L)
                    sh = pl.ds(D2 + L * g, L)
                    for rows, pk in ((rows_s[p], pk_s[p]),
                                     (rows_d[p], pk_d[p])):
                        ua = lax.bitcast_convert_type(rows[r, sl],
                                                      jnp.uint32)
                        ub = lax.bitcast_convert_type(rows[r, sh],
                                                      jnp.uint32)
                        s16 = jnp.uint32(16)
                        ra = lax.shift_right_logical(
                            ua + half +
                            (lax.shift_right_logical(ua, s16) & one), s16)
                        rb = (ub + half +
                              (lax.shift_right_logical(ub, s16) & one)
                              ) & hmask
                        pk[r, sl] = lax.bitcast_convert_type(
                            ra | rb, jnp.int32)
                return carry

            lax.fori_loop(0, C, row, 0)

        def tick(c, p):
            q = 1 - p
            wait_idx(q)

            @pl.when(c >= 1)
            def _(): wait_scatter(q)

            issue_gather(q)
            wait_gather(p)

            @pl.when(c + 2 < n)
            def _(): issue_idx(c + 2, p)

            pack_rows(p)
            issue_scatter(c, p)

        def tick_static(c):
            p = c % 2
            q = 1 - p
            if c + 1 < n:
                wait_idx(q)
            if c >= 1:
                wait_scatter(q)
            if c + 1 < n:
                issue_gather(q)
            wait_gather(p)
            if c + 2 < n:
                issue_idx(c + 2, p)
            pack_rows(p)
            issue_scatter(c, p)

        issue_idx(0, 0)
        issue_idx(1, 1)
        wait_idx(0)
        issue_gather(0)

        nloop = 2 * ((n - 2) // 2)

        def pair(oi, carry):
            tick(2 * oi, 0)
            tick(2 * oi + 1, 1)
            return carry

        lax.fori_loop(0, nloop // 2, pair, 0)
        for c in range(nloop, n):
            tick_static(c)
        wait_scatter((n - 1) % 2)

    return sc_gather


def _unpack_bf16_halves(words):
    lo = lax.bitcast_convert_type(jnp.left_shift(words, 16), jnp.float32)
    hi = lax.bitcast_convert_type(
        jnp.bitwise_and(words, jnp.int32(-65536)), jnp.float32)
    return jnp.concatenate([lo, hi], axis=1)


def _tc_body(hs_ref, hd_ref, e_ref, g_ref, b_ref, w1_ref, b1_ref,
             w2_ref, b2_ref, out_ref, *, IN):
    hs = _unpack_bf16_halves(hs_ref[...])
    hd = _unpack_bf16_halves(hd_ref[...])
    e = e_ref[...]
    pair = jnp.concatenate([hs + hd, hs * hd, e], axis=1)
    inv = 1.0 / IN
    mu = jnp.sum(pair, axis=1, keepdims=True) * inv
    cen = pair - mu
    var = jnp.sum(cen * cen, axis=1, keepdims=True) * inv
    pairn = cen * lax.rsqrt(var + 1e-5) * g_ref[...] + b_ref[...]
    h = jnp.dot(pairn.astype(jnp.bfloat16), w1_ref[...],
                preferred_element_type=jnp.float32)
    h = h + b1_ref[...]
    h = 0.5 * h * (1.0 + lax.erf(h * 0.7071067811865476))
    delta = jnp.dot(h.astype(jnp.bfloat16), w2_ref[...],
                    preferred_element_type=jnp.float32)
    out_ref[...] = e + delta + b2_ref[...]


def _tc_body_carry(carry_ref, *rest, IN):
    _tc_body(*rest, IN=IN)


@functools.lru_cache(maxsize=None)
def _make_tc_chunk(E, Ec, D, IN, HID, k, with_carry, dtype_name,
                   interpret=False):
    dtype = jnp.dtype(dtype_name)
    BE = 6400
    assert Ec % BE == 0
    nb = Ec // BE
    off = k * nb

    def chunk_blk(i):
        return (i, 0)

    def full_blk(i):
        return (i + off, 0)

    def fixed(i):
        return (0, 0)

    in_specs = [
        pl.BlockSpec((BE, D // 2), chunk_blk),
        pl.BlockSpec((BE, D // 2), chunk_blk),
        pl.BlockSpec((BE, D), full_blk),
        pl.BlockSpec((1, IN), fixed),
        pl.BlockSpec((1, IN), fixed),
        pl.BlockSpec((IN, HID), fixed),
        pl.BlockSpec((1, HID), fixed),
        pl.BlockSpec((HID, D), fixed),
        pl.BlockSpec((1, D), fixed),
    ]
    body = functools.partial(_tc_body, IN=IN)
    aliases = {}
    if with_carry:
        in_specs = [pl.BlockSpec(memory_space=pl.ANY)] + in_specs
        body = functools.partial(_tc_body_carry, IN=IN)
        aliases = {0: 0}

    return pl.pallas_call(
        body,
        grid=(nb,),
        in_specs=in_specs,
        out_specs=pl.BlockSpec((BE, D), full_blk),
        out_shape=jax.ShapeDtypeStruct((E, D), dtype),
        input_output_aliases=aliases,
        interpret=interpret,
    )


def kernel(x, edge_index, edge_attr, ln_gamma, ln_beta, W1, b1, W2, b2):
    N, D = x.shape
    E = edge_attr.shape[0]
    IN, HID = W1.shape
    src = edge_index[0]
    dst = edge_index[1]
    K = 5
    Ec = E // K
    assert E % K == 0
    sc_gather = _make_sc_gather(N, Ec, D, x.dtype.name)
    g2 = ln_gamma.reshape(1, IN)
    bt = ln_beta.reshape(1, IN)
    w1b = W1.astype(jnp.bfloat16)
    b1r = b1.reshape(1, HID)
    w2b = W2.astype(jnp.bfloat16)
    b2r = b2.reshape(1, D)
    out = None
    for k in range(K):
        s_k = lax.slice_in_dim(src, k * Ec, (k + 1) * Ec)
        d_k = lax.slice_in_dim(dst, k * Ec, (k + 1) * Ec)
        hs_k, hd_k = sc_gather(x, s_k, d_k)
        tc = _make_tc_chunk(E, Ec, D, IN, HID, k, k > 0, x.dtype.name)
        args = (hs_k, hd_k, edge_attr, g2, bt, w1b, b1r, w2b, b2r)
        out = tc(*args) if k == 0 else tc(out, *args)
    return out


# speedup vs baseline: 1.9490x; 1.9490x over previous
import functools

import jax
import jax.numpy as jnp
from jax import lax
from jax.experimental import pallas as pl
from jax.experimental.pallas import tpu as pltpu
from jax.experimental.pallas import tpu_sc as plsc


@functools.lru_cache(maxsize=None)
def _make_sc_gather(N, Ec, D, dtype_name):
    dtype = jnp.dtype(dtype_name)
    info = plsc.get_sparse_core_info()
    NC, NS = info.num_cores, info.num_subcores
    NW = NC * NS
    assert Ec % NW == 0
    epw = Ec // NW
    C = next(c for c in range(64, 7, -8) if epw % c == 0)
    n = epw // C
    assert n >= 4
    SROW = next(s for s in (16, 10, 8, 5, 4, 2, 1)
                if N % s == 0 and (N // s) % 8 == 0)
    rpt = N // SROW

    mesh = plsc.VectorSubcoreMesh(core_axis_name="c", subcore_axis_name="s")

    @functools.partial(
        pl.kernel,
        out_type=(
            jax.ShapeDtypeStruct((Ec, D), dtype),
            jax.ShapeDtypeStruct((Ec, D), dtype),
        ),
        mesh=mesh,
        scratch_types=[
            pltpu.VMEM((C,), jnp.int32),
            pltpu.VMEM((C,), jnp.int32),
            pltpu.VMEM((C,), jnp.int32),
            pltpu.VMEM((C,), jnp.int32),
            pltpu.VMEM((C, D), dtype),
            pltpu.VMEM((C, D), dtype),
            pltpu.VMEM((C, D), dtype),
            pltpu.VMEM((C, D), dtype),
            pltpu.VMEM_SHARED((N, D), dtype),
            pltpu.SemaphoreType.DMA,
            pltpu.SemaphoreType.DMA,
            pltpu.SemaphoreType.DMA,
            pltpu.SemaphoreType.DMA,
            pltpu.SemaphoreType.DMA,
            pltpu.SemaphoreType.DMA,
        ],
    )
    def sc_gather(x_hbm, src_hbm, dst_hbm, hs_hbm, hd_hbm,
                  i_s0, i_s1, i_d0, i_d1, r_s0, r_s1, r_d0, r_d1, x_sp,
                  sem_i0, sem_i1, sem_g0, sem_g1, sem_v0, sem_v1):
        idx_s = (i_s0, i_s1)
        idx_d = (i_d0, i_d1)
        rows_s = (r_s0, r_s1)
        rows_d = (r_d0, r_d1)
        sem_i = (sem_i0, sem_i1)
        sem_g = (sem_g0, sem_g1)
        sem_v = (sem_v0, sem_v1)
        cid = lax.axis_index("c")
        sid = lax.axis_index("s")
        wid = cid * NS + sid
        base_w = wid * epw

        @pl.when(sid < SROW)
        def _stage():
            pltpu.sync_copy(x_hbm.at[pl.ds(sid * rpt, rpt)],
                            x_sp.at[pl.ds(sid * rpt, rpt)])

        plsc.subcore_barrier()

        def issue_idx(c, p):
            base = base_w + c * C
            pltpu.async_copy(src_hbm.at[pl.ds(base, C)], idx_s[p], sem_i[p])
            pltpu.async_copy(dst_hbm.at[pl.ds(base, C)], idx_d[p], sem_i[p])

        def wait_idx(p):
            pltpu.make_async_copy(
                src_hbm.at[pl.ds(0, C)], idx_s[p], sem_i[p]).wait()
            pltpu.make_async_copy(
                dst_hbm.at[pl.ds(0, C)], idx_d[p], sem_i[p]).wait()

        def issue_gather(p):
            pltpu.async_copy(x_sp.at[idx_s[p]], rows_s[p], sem_g[p])
            pltpu.async_copy(x_sp.at[idx_d[p]], rows_d[p], sem_g[p])

        def wait_gather(p):
            pltpu.make_async_copy(
                x_sp.at[idx_s[p]], rows_s[p], sem_g[p]).wait()
            pltpu.make_async_copy(
                x_sp.at[idx_d[p]], rows_d[p], sem_g[p]).wait()

        def issue_scatter(c, p):
            base = base_w + c * C
            pltpu.async_copy(rows_s[p], hs_hbm.at[pl.ds(base, C)], sem_v[p])
            pltpu.async_copy(rows_d[p], hd_hbm.at[pl.ds(base, C)], sem_v[p])

        def wait_scatter(p):
            pltpu.make_async_copy(
                rows_s[p], hs_hbm.at[pl.ds(0, C)], sem_v[p]).wait()
            pltpu.make_async_copy(
                rows_d[p], hd_hbm.at[pl.ds(0, C)], sem_v[p]).wait()

        def tick(c, p):
            q = 1 - p
            wait_idx(q)

            @pl.when(c >= 1)
            def _(): wait_scatter(q)

            issue_gather(q)
            wait_gather(p)

            @pl.when(c + 2 < n)
            def _(): issue_idx(c + 2, p)

            issue_scatter(c, p)

        def tick_static(c):
            p = c % 2
            q = 1 - p
            if c + 1 < n:
                wait_idx(q)
            if c >= 1:
                wait_scatter(q)
            if c + 1 < n:
                issue_gather(q)
            wait_gather(p)
            if c + 2 < n:
                issue_idx(c + 2, p)
            issue_scatter(c, p)

        issue_idx(0, 0)
        issue_idx(1, 1)
        wait_idx(0)
        issue_gather(0)

        nloop = 2 * ((n - 2) // 2)

        def pair(oi, carry):
            tick(2 * oi, 0)
            tick(2 * oi + 1, 1)
            return carry

        lax.fori_loop(0, nloop // 2, pair, 0)
        for c in range(nloop, n):
            tick_static(c)
        wait_scatter((n - 1) % 2)

    return sc_gather


def _tc_body(hs_ref, hd_ref, e_ref, g_ref, b_ref, w1_ref, b1_ref,
             w2_ref, b2_ref, out_ref, *, IN):
    hs = hs_ref[...]
    hd = hd_ref[...]
    e = e_ref[...]
    pair = jnp.concatenate([hs + hd, hs * hd, e], axis=1)
    inv = 1.0 / IN
    mu = jnp.sum(pair, axis=1, keepdims=True) * inv
    cen = pair - mu
    var = jnp.sum(cen * cen, axis=1, keepdims=True) * inv
    pairn = cen * lax.rsqrt(var + 1e-5) * g_ref[...] + b_ref[...]
    h = jnp.dot(pairn.astype(jnp.bfloat16), w1_ref[...],
                preferred_element_type=jnp.float32)
    h = h + b1_ref[...]
    h = 0.5 * h * (1.0 + lax.erf(h * 0.7071067811865476))
    delta = jnp.dot(h.astype(jnp.bfloat16), w2_ref[...],
                    preferred_element_type=jnp.float32)
    out_ref[...] = e + delta + b2_ref[...]


def _tc_body_carry(carry_ref, *rest, IN):
    _tc_body(*rest, IN=IN)


@functools.lru_cache(maxsize=None)
def _make_tc_chunk(E, Ec, D, IN, HID, k, with_carry, dtype_name,
                   interpret=False):
    dtype = jnp.dtype(dtype_name)
    BE = 6400
    assert Ec % BE == 0
    nb = Ec // BE
    off = k * nb

    def chunk_blk(i):
        return (i, 0)

    def full_blk(i):
        return (i + off, 0)

    def fixed(i):
        return (0, 0)

    in_specs = [
        pl.BlockSpec((BE, D), chunk_blk),
        pl.BlockSpec((BE, D), chunk_blk),
        pl.BlockSpec((BE, D), full_blk),
        pl.BlockSpec((1, IN), fixed),
        pl.BlockSpec((1, IN), fixed),
        pl.BlockSpec((IN, HID), fixed),
        pl.BlockSpec((1, HID), fixed),
        pl.BlockSpec((HID, D), fixed),
        pl.BlockSpec((1, D), fixed),
    ]
    body = functools.partial(_tc_body, IN=IN)
    aliases = {}
    if with_carry:
        in_specs = [pl.BlockSpec(memory_space=pl.ANY)] + in_specs
        body = functools.partial(_tc_body_carry, IN=IN)
        aliases = {0: 0}

    return pl.pallas_call(
        body,
        grid=(nb,),
        in_specs=in_specs,
        out_specs=pl.BlockSpec((BE, D), full_blk),
        out_shape=jax.ShapeDtypeStruct((E, D), dtype),
        input_output_aliases=aliases,
        interpret=interpret,
    )


def kernel(x, edge_index, edge_attr, ln_gamma, ln_beta, W1, b1, W2, b2):
    N, D = x.shape
    E = edge_attr.shape[0]
    IN, HID = W1.shape
    src = edge_index[0]
    dst = edge_index[1]
    K = 5
    Ec = E // K
    assert E % K == 0
    sc_gather = _make_sc_gather(N, Ec, D, x.dtype.name)
    g2 = ln_gamma.reshape(1, IN)
    bt = ln_beta.reshape(1, IN)
    w1b = W1.astype(jnp.bfloat16)
    b1r = b1.reshape(1, HID)
    w2b = W2.astype(jnp.bfloat16)
    b2r = b2.reshape(1, D)
    out = None
    for k in range(K):
        s_k = lax.slice_in_dim(src, k * Ec, (k + 1) * Ec)
        d_k = lax.slice_in_dim(dst, k * Ec, (k + 1) * Ec)
        hs_k, hd_k = sc_gather(x, s_k, d_k)
        tc = _make_tc_chunk(E, Ec, D, IN, HID, k, k > 0, x.dtype.name)
        args = (hs_k, hd_k, edge_attr, g2, bt, w1b, b1r, w2b, b2r)
        out = tc(*args) if k == 0 else tc(out, *args)
    return out


# speedup vs baseline: 1.9511x; 1.0011x over previous
import functools

import jax
import jax.numpy as jnp
from jax import lax
from jax.experimental import pallas as pl
from jax.experimental.pallas import tpu as pltpu
from jax.experimental.pallas import tpu_sc as plsc


@functools.lru_cache(maxsize=None)
def _make_sc_gather(N, Ec, D, dtype_name):
    dtype = jnp.dtype(dtype_name)
    info = plsc.get_sparse_core_info()
    NC, NS = info.num_cores, info.num_subcores
    NW = NC * NS
    assert Ec % NW == 0
    epw = Ec // NW
    C = next(c for c in range(64, 7, -8) if epw % c == 0)
    n = epw // C
    assert n >= 4
    rpt = (N // NS) // 8 * 8
    rlast = N - (NS - 1) * rpt

    mesh = plsc.VectorSubcoreMesh(core_axis_name="c", subcore_axis_name="s")

    @functools.partial(
        pl.kernel,
        out_type=(
            jax.ShapeDtypeStruct((Ec, D), dtype),
            jax.ShapeDtypeStruct((Ec, D), dtype),
        ),
        mesh=mesh,
        scratch_types=[
            pltpu.VMEM((C,), jnp.int32),
            pltpu.VMEM((C,), jnp.int32),
            pltpu.VMEM((C,), jnp.int32),
            pltpu.VMEM((C,), jnp.int32),
            pltpu.VMEM((C, D), dtype),
            pltpu.VMEM((C, D), dtype),
            pltpu.VMEM((C, D), dtype),
            pltpu.VMEM((C, D), dtype),
            pltpu.VMEM_SHARED((N, D), dtype),
            pltpu.SemaphoreType.DMA,
            pltpu.SemaphoreType.DMA,
            pltpu.SemaphoreType.DMA,
            pltpu.SemaphoreType.DMA,
            pltpu.SemaphoreType.DMA,
            pltpu.SemaphoreType.DMA,
        ],
    )
    def sc_gather(x_hbm, src_hbm, dst_hbm, hs_hbm, hd_hbm,
                  i_s0, i_s1, i_d0, i_d1, r_s0, r_s1, r_d0, r_d1, x_sp,
                  sem_i0, sem_i1, sem_g0, sem_g1, sem_v0, sem_v1):
        idx_s = (i_s0, i_s1)
        idx_d = (i_d0, i_d1)
        rows_s = (r_s0, r_s1)
        rows_d = (r_d0, r_d1)
        sem_i = (sem_i0, sem_i1)
        sem_g = (sem_g0, sem_g1)
        sem_v = (sem_v0, sem_v1)
        cid = lax.axis_index("c")
        sid = lax.axis_index("s")
        wid = cid * NS + sid
        base_w = wid * epw

        def issue_idx(c, p):
            base = base_w + c * C
            pltpu.async_copy(src_hbm.at[pl.ds(base, C)], idx_s[p], sem_i[p])
            pltpu.async_copy(dst_hbm.at[pl.ds(base, C)], idx_d[p], sem_i[p])

        def wait_idx(p):
            pltpu.make_async_copy(
                src_hbm.at[pl.ds(0, C)], idx_s[p], sem_i[p]).wait()
            pltpu.make_async_copy(
                dst_hbm.at[pl.ds(0, C)], idx_d[p], sem_i[p]).wait()

        def issue_gather(p):
            pltpu.async_copy(x_sp.at[idx_s[p]], rows_s[p], sem_g[p])
            pltpu.async_copy(x_sp.at[idx_d[p]], rows_d[p], sem_g[p])

        def wait_gather(p):
            pltpu.make_async_copy(
                x_sp.at[idx_s[p]], rows_s[p], sem_g[p]).wait()
            pltpu.make_async_copy(
                x_sp.at[idx_d[p]], rows_d[p], sem_g[p]).wait()

        def issue_scatter(c, p):
            base = base_w + c * C
            pltpu.async_copy(rows_s[p], hs_hbm.at[pl.ds(base, C)], sem_v[p])
            pltpu.async_copy(rows_d[p], hd_hbm.at[pl.ds(base, C)], sem_v[p])

        def wait_scatter(p):
            pltpu.make_async_copy(
                rows_s[p], hs_hbm.at[pl.ds(0, C)], sem_v[p]).wait()
            pltpu.make_async_copy(
                rows_d[p], hd_hbm.at[pl.ds(0, C)], sem_v[p]).wait()

        def tick(c, p):
            q = 1 - p
            wait_idx(q)

            @pl.when(c >= 1)
            def _(): wait_scatter(q)

            issue_gather(q)
            wait_gather(p)

            @pl.when(c + 2 < n)
            def _(): issue_idx(c + 2, p)

            issue_scatter(c, p)

        def tick_static(c):
            p = c % 2
            q = 1 - p
            if c + 1 < n:
                wait_idx(q)
            if c >= 1:
                wait_scatter(q)
            if c + 1 < n:
                issue_gather(q)
            wait_gather(p)
            if c + 2 < n:
                issue_idx(c + 2, p)
            issue_scatter(c, p)

        issue_idx(0, 0)
        issue_idx(1, 1)

        if rpt > 0:
            @pl.when(sid < NS - 1)
            def _stage():
                pltpu.sync_copy(x_hbm.at[pl.ds(sid * rpt, rpt)],
                                x_sp.at[pl.ds(sid * rpt, rpt)])

        @pl.when(sid == NS - 1)
        def _stage_last():
            pltpu.sync_copy(x_hbm.at[pl.ds((NS - 1) * rpt, rlast)],
                            x_sp.at[pl.ds((NS - 1) * rpt, rlast)])

        plsc.subcore_barrier()

        wait_idx(0)
        issue_gather(0)

        nloop = 2 * ((n - 2) // 2)

        def pair(oi, carry):
            tick(2 * oi, 0)
            tick(2 * oi + 1, 1)
            return carry

        lax.fori_loop(0, nloop // 2, pair, 0)
        for c in range(nloop, n):
            tick_static(c)
        wait_scatter((n - 1) % 2)

    return sc_gather


def _tc_body(hs_ref, hd_ref, e_ref, g_ref, b_ref, w1_ref, b1_ref,
             w2_ref, b2_ref, out_ref, *, IN):
    hs = hs_ref[...]
    hd = hd_ref[...]
    e = e_ref[...]
    pair = jnp.concatenate([hs + hd, hs * hd, e], axis=1)
    inv = 1.0 / IN
    mu = jnp.sum(pair, axis=1, keepdims=True) * inv
    cen = pair - mu
    var = jnp.sum(cen * cen, axis=1, keepdims=True) * inv
    pairn = cen * lax.rsqrt(var + 1e-5) * g_ref[...] + b_ref[...]
    h = jnp.dot(pairn.astype(jnp.bfloat16), w1_ref[...],
                preferred_element_type=jnp.float32)
    h = h + b1_ref[...]
    h = 0.5 * h * (1.0 + lax.erf(h * 0.7071067811865476))
    delta = jnp.dot(h.astype(jnp.bfloat16), w2_ref[...],
                    preferred_element_type=jnp.float32)
    out_ref[...] = e + delta + b2_ref[...]


def _tc_body_carry(carry_ref, *rest, IN):
    _tc_body(*rest, IN=IN)


@functools.lru_cache(maxsize=None)
def _make_tc_chunk(E, Ec, D, IN, HID, k, with_carry, dtype_name,
                   interpret=False):
    dtype = jnp.dtype(dtype_name)
    BE = 6400
    assert Ec % BE == 0
    nb = Ec // BE
    off = k * nb

    def chunk_blk(i):
        return (i, 0)

    def full_blk(i):
        return (i + off, 0)

    def fixed(i):
        return (0, 0)

    in_specs = [
        pl.BlockSpec((BE, D), chunk_blk),
        pl.BlockSpec((BE, D), chunk_blk),
        pl.BlockSpec((BE, D), full_blk),
        pl.BlockSpec((1, IN), fixed),
        pl.BlockSpec((1, IN), fixed),
        pl.BlockSpec((IN, HID), fixed),
        pl.BlockSpec((1, HID), fixed),
        pl.BlockSpec((HID, D), fixed),
        pl.BlockSpec((1, D), fixed),
    ]
    body = functools.partial(_tc_body, IN=IN)
    aliases = {}
    if with_carry:
        in_specs = [pl.BlockSpec(memory_space=pl.ANY)] + in_specs
        body = functools.partial(_tc_body_carry, IN=IN)
        aliases = {0: 0}

    return pl.pallas_call(
        body,
        grid=(nb,),
        in_specs=in_specs,
        out_specs=pl.BlockSpec((BE, D), full_blk),
        out_shape=jax.ShapeDtypeStruct((E, D), dtype),
        input_output_aliases=aliases,
        interpret=interpret,
    )


def kernel(x, edge_index, edge_attr, ln_gamma, ln_beta, W1, b1, W2, b2):
    N, D = x.shape
    E = edge_attr.shape[0]
    IN, HID = W1.shape
    src = edge_index[0]
    dst = edge_index[1]
    K = 5
    Ec = E // K
    assert E % K == 0
    sc_gather = _make_sc_gather(N, Ec, D, x.dtype.name)
    g2 = ln_gamma.reshape(1, IN)
    bt = ln_beta.reshape(1, IN)
    w1b = W1.astype(jnp.bfloat16)
    b1r = b1.reshape(1, HID)
    w2b = W2.astype(jnp.bfloat16)
    b2r = b2.reshape(1, D)
    out = None
    for k in range(K):
        s_k = lax.slice_in_dim(src, k * Ec, (k + 1) * Ec)
        d_k = lax.slice_in_dim(dst, k * Ec, (k + 1) * Ec)
        hs_k, hd_k = sc_gather(x, s_k, d_k)
        tc = _make_tc_chunk(E, Ec, D, IN, HID, k, k > 0, x.dtype.name)
        args = (hs_k, hd_k, edge_attr, g2, bt, w1b, b1r, w2b, b2r)
        out = tc(*args) if k == 0 else tc(out, *args)
    return out
